# baseline (device time: 14564 ns/iter reference)
import jax
import jax.numpy as jnp
from jax import lax
from jax.experimental import pallas as pl
from jax.experimental.pallas import tpu as pltpu

N_CHUNK = 4


def kernel(partial, resid, gamma):
    _, m, d = partial.shape
    half = m // 2
    ch = half // N_CHUNK

    def body(p_hbm, r_hbm, g_hbm, out_ref, p_loc, r_loc, g_loc, send_a, recv_a,
             p_sems, rg_sems, send_sems_a, recv_sems_a, send_sems_b, recv_sems_b):
        my_x = lax.axis_index("x")
        my_y = lax.axis_index("y")
        x_nbr = (1 - my_x, my_y)
        y_nbr = (my_x, 1 - my_y)
        my_base = my_y * half
        other_base = (1 - my_y) * half

        def p_dma(c):
            sl = pl.ds(c * ch, ch)
            return pltpu.make_async_copy(
                p_hbm.at[0, pl.ds(my_base + c * ch, ch)], p_loc.at[sl],
                p_sems.at[c])

        r_dma = pltpu.make_async_copy(
            r_hbm.at[pl.ds(my_base, half)], r_loc, rg_sems.at[0])
        g_dma = pltpu.make_async_copy(g_hbm, g_loc, rg_sems.at[1])
        for c in range(N_CHUNK):
            p_dma(c).start()
        r_dma.start()
        g_dma.start()

        barrier = pltpu.get_barrier_semaphore()
        for nbr in (x_nbr, y_nbr):
            pl.semaphore_signal(
                barrier, inc=1, device_id=nbr,
                device_id_type=pl.DeviceIdType.MESH,
            )
        pl.semaphore_wait(barrier, 2)

        def a_rdma(c):
            sl = pl.ds(c * ch, ch)
            return pltpu.make_async_remote_copy(
                src_ref=send_a.at[sl],
                dst_ref=recv_a.at[sl],
                send_sem=send_sems_a.at[c],
                recv_sem=recv_sems_a.at[c],
                device_id=x_nbr,
                device_id_type=pl.DeviceIdType.MESH,
            )

        def b_rdma(c, base):
            sl = pl.ds(base + c * ch, ch)
            return pltpu.make_async_remote_copy(
                src_ref=out_ref.at[sl],
                dst_ref=out_ref.at[sl],
                send_sem=send_sems_b.at[c],
                recv_sem=recv_sems_b.at[c],
                device_id=y_nbr,
                device_id_type=pl.DeviceIdType.MESH,
            )

        for c in range(N_CHUNK):
            sl = pl.ds(c * ch, ch)
            p_dma(c).wait()
            send_a[sl] = p_loc[sl].astype(jnp.bfloat16)
            a_rdma(c).start()

        r_dma.wait()
        g_dma.wait()

        for c in range(N_CHUNK):
            a_rdma(c).wait_recv()
            sl = pl.ds(c * ch, ch)
            t = (p_loc[sl] + recv_a[sl].astype(jnp.float32) + r_loc[sl])
            inv = lax.rsqrt(jnp.mean(t * t, axis=-1, keepdims=True) + 1e-6)
            out_ref[pl.ds(my_base + c * ch, ch)] = (
                t * inv * g_loc[...]).astype(jnp.bfloat16)
            b_rdma(c, my_base).start()

        for c in range(N_CHUNK):
            b_rdma(c, other_base).wait_recv()
        for c in range(N_CHUNK):
            a_rdma(c).wait_send()
            b_rdma(c, my_base).wait_send()

    return pl.pallas_call(
        body,
        out_shape=jax.ShapeDtypeStruct((m, d), jnp.bfloat16),
        in_specs=[
            pl.BlockSpec(memory_space=pltpu.HBM),
            pl.BlockSpec(memory_space=pltpu.HBM),
            pl.BlockSpec(memory_space=pltpu.HBM),
        ],
        out_specs=pl.BlockSpec(memory_space=pltpu.VMEM),
        scratch_shapes=[
            pltpu.VMEM((half, d), jnp.float32),
            pltpu.VMEM((half, d), jnp.float32),
            pltpu.VMEM((1, d), jnp.float32),
            pltpu.VMEM((half, d), jnp.bfloat16),
            pltpu.VMEM((half, d), jnp.bfloat16),
            pltpu.SemaphoreType.DMA((N_CHUNK,)),
            pltpu.SemaphoreType.DMA((2,)),
            pltpu.SemaphoreType.DMA((N_CHUNK,)),
            pltpu.SemaphoreType.DMA((N_CHUNK,)),
            pltpu.SemaphoreType.DMA((N_CHUNK,)),
            pltpu.SemaphoreType.DMA((N_CHUNK,)),
        ],
        compiler_params=pltpu.CompilerParams(collective_id=0),
    )(partial, resid, gamma.reshape(1, d))


# device time: 13616 ns/iter; 1.0696x vs baseline; 1.0696x over previous
import jax
import jax.numpy as jnp
from jax import lax
from jax.experimental import pallas as pl
from jax.experimental.pallas import tpu as pltpu

N_CHUNK = 8


def kernel(partial, resid, gamma):
    _, m, d = partial.shape
    half = m // 2
    ch = half // N_CHUNK

    def body(p_ref, r_ref, g_ref, out_ref, send_a, recv_a,
             send_sems_a, recv_sems_a, send_sems_b, recv_sems_b):
        my_x = lax.axis_index("x")
        my_y = lax.axis_index("y")
        x_nbr = (1 - my_x, my_y)
        y_nbr = (my_x, 1 - my_y)
        my_base = my_y * half
        other_base = (1 - my_y) * half

        barrier = pltpu.get_barrier_semaphore()
        for nbr in (x_nbr, y_nbr):
            pl.semaphore_signal(
                barrier, inc=1, device_id=nbr,
                device_id_type=pl.DeviceIdType.MESH,
            )
        pl.semaphore_wait(barrier, 2)

        def a_rdma(c):
            sl = pl.ds(c * ch, ch)
            return pltpu.make_async_remote_copy(
                src_ref=send_a.at[sl],
                dst_ref=recv_a.at[sl],
                send_sem=send_sems_a.at[c],
                recv_sem=recv_sems_a.at[c],
                device_id=x_nbr,
                device_id_type=pl.DeviceIdType.MESH,
            )

        def b_rdma(c, base):
            sl = pl.ds(base + c * ch, ch)
            return pltpu.make_async_remote_copy(
                src_ref=out_ref.at[sl],
                dst_ref=out_ref.at[sl],
                send_sem=send_sems_b.at[c],
                recv_sem=recv_sems_b.at[c],
                device_id=y_nbr,
                device_id_type=pl.DeviceIdType.MESH,
            )

        for c in range(N_CHUNK):
            sl = pl.ds(c * ch, ch)
            send_a[sl] = p_ref[0, pl.ds(my_base + c * ch, ch)].astype(jnp.bfloat16)
            a_rdma(c).start()

        for c in range(N_CHUNK):
            a_rdma(c).wait_recv()
            sl = pl.ds(c * ch, ch)
            rows = pl.ds(my_base + c * ch, ch)
            t = (send_a[sl].astype(jnp.float32)
                 + recv_a[sl].astype(jnp.float32)
                 + r_ref[rows])
            inv = lax.rsqrt(jnp.mean(t * t, axis=-1, keepdims=True) + 1e-6)
            out_ref[rows] = (t * inv * g_ref[...]).astype(jnp.bfloat16)
            b_rdma(c, my_base).start()

        for c in range(N_CHUNK):
            b_rdma(c, other_base).wait_recv()
        for c in range(N_CHUNK):
            a_rdma(c).wait_send()
            b_rdma(c, my_base).wait_send()

    return pl.pallas_call(
        body,
        out_shape=jax.ShapeDtypeStruct((m, d), jnp.bfloat16),
        in_specs=[
            pl.BlockSpec(memory_space=pltpu.VMEM),
            pl.BlockSpec(memory_space=pltpu.VMEM),
            pl.BlockSpec(memory_space=pltpu.VMEM),
        ],
        out_specs=pl.BlockSpec(memory_space=pltpu.VMEM),
        scratch_shapes=[
            pltpu.VMEM((half, d), jnp.bfloat16),
            pltpu.VMEM((half, d), jnp.bfloat16),
            pltpu.SemaphoreType.DMA((N_CHUNK,)),
            pltpu.SemaphoreType.DMA((N_CHUNK,)),
            pltpu.SemaphoreType.DMA((N_CHUNK,)),
            pltpu.SemaphoreType.DMA((N_CHUNK,)),
        ],
        compiler_params=pltpu.CompilerParams(collective_id=0),
    )(partial, resid, gamma.reshape(1, d))


# device time: 13132 ns/iter; 1.1090x vs baseline; 1.0369x over previous
import jax
import jax.numpy as jnp
from jax import lax
from jax.experimental import pallas as pl
from jax.experimental.pallas import tpu as pltpu

CH = 32
B_CHUNKS = 5
O_CHUNKS = 6
N_CHUNK = B_CHUNKS + O_CHUNKS


def kernel(partial, resid, gamma):
    _, m, d = partial.shape
    s_rows = N_CHUNK * CH
    b_rows = B_CHUNKS * CH
    assert 2 * s_rows - m == O_CHUNKS * CH

    def body(p_ref, r_ref, g_ref, out_ref, send_a, recv_a,
             send_sems_a, recv_sems_a, send_sems_b, recv_sems_b):
        my_x = lax.axis_index("x")
        my_y = lax.axis_index("y")
        x_nbr = (1 - my_x, my_y)
        y_nbr = (my_x, 1 - my_y)
        b_base = my_y * (m - b_rows)
        brecv_base = (1 - my_y) * (m - b_rows)

        def grow(c):
            if c < B_CHUNKS:
                return b_base + c * CH
            return b_rows + (c - B_CHUNKS) * CH

        for c in range(N_CHUNK):
            send_a[pl.ds(c * CH, CH)] = (
                p_ref[0, pl.ds(grow(c), CH)].astype(jnp.bfloat16))

        barrier = pltpu.get_barrier_semaphore()
        for nbr in (x_nbr, y_nbr):
            pl.semaphore_signal(
                barrier, inc=1, device_id=nbr,
                device_id_type=pl.DeviceIdType.MESH,
            )
        pl.semaphore_wait(barrier, 2)

        def a_rdma(c):
            sl = pl.ds(c * CH, CH)
            return pltpu.make_async_remote_copy(
                src_ref=send_a.at[sl],
                dst_ref=recv_a.at[sl],
                send_sem=send_sems_a.at[c],
                recv_sem=recv_sems_a.at[c],
                device_id=x_nbr,
                device_id_type=pl.DeviceIdType.MESH,
            )

        def b_rdma(c, base):
            sl = pl.ds(base + c * CH, CH)
            return pltpu.make_async_remote_copy(
                src_ref=out_ref.at[sl],
                dst_ref=out_ref.at[sl],
                send_sem=send_sems_b.at[c],
                recv_sem=recv_sems_b.at[c],
                device_id=y_nbr,
                device_id_type=pl.DeviceIdType.MESH,
            )

        for c in range(N_CHUNK):
            a_rdma(c).start()

        for c in range(N_CHUNK):
            a_rdma(c).wait_recv()
            sl = pl.ds(c * CH, CH)
            rows = pl.ds(grow(c), CH)
            t = (send_a[sl].astype(jnp.float32)
                 + recv_a[sl].astype(jnp.float32)
                 + r_ref[rows])
            inv = lax.rsqrt(jnp.mean(t * t, axis=-1, keepdims=True) + 1e-6)
            out_ref[rows] = (t * inv * g_ref[...]).astype(jnp.bfloat16)
            if c < B_CHUNKS:
                b_rdma(c, b_base).start()

        for c in range(B_CHUNKS):
            b_rdma(c, brecv_base).wait_recv()
        for c in range(N_CHUNK):
            a_rdma(c).wait_send()
        for c in range(B_CHUNKS):
            b_rdma(c, b_base).wait_send()

    return pl.pallas_call(
        body,
        out_shape=jax.ShapeDtypeStruct((m, d), jnp.bfloat16),
        in_specs=[
            pl.BlockSpec(memory_space=pltpu.VMEM),
            pl.BlockSpec(memory_space=pltpu.VMEM),
            pl.BlockSpec(memory_space=pltpu.VMEM),
        ],
        out_specs=pl.BlockSpec(memory_space=pltpu.VMEM),
        scratch_shapes=[
            pltpu.VMEM((N_CHUNK * CH, d), jnp.bfloat16),
            pltpu.VMEM((N_CHUNK * CH, d), jnp.bfloat16),
            pltpu.SemaphoreType.DMA((N_CHUNK,)),
            pltpu.SemaphoreType.DMA((N_CHUNK,)),
            pltpu.SemaphoreType.DMA((B_CHUNKS,)),
            pltpu.SemaphoreType.DMA((B_CHUNKS,)),
        ],
        compiler_params=pltpu.CompilerParams(collective_id=0),
    )(partial, resid, gamma.reshape(1, d))
